# Initial kernel scaffold; baseline (speedup 1.0000x reference)
import jax
import jax.numpy as jnp
from jax.experimental import pallas as pl


def kernel(data, ivectors, ovectors):
    raise NotImplementedError("write your pallas kernel here")


# speedup vs baseline: 1.6832x; 1.6832x over previous
import functools

import jax
import jax.numpy as jnp
from jax import lax
from jax.experimental import pallas as pl
from jax.experimental.pallas import tpu as pltpu
from jax.experimental.pallas import tpu_sc as plsc

N_NODES_P1 = 1000001
DIM = 64
BATCH = 16384
HIST = 50

B = BATCH * HIST
NW = 32
B_PER_W = B // NW
CHUNK = 128
N_CHUNKS = B_PER_W // CHUNK


def _make_gather():
    mesh = plsc.VectorSubcoreMesh(core_axis_name="c", subcore_axis_name="s")

    @functools.partial(
        pl.kernel,
        mesh=mesh,
        out_type=jax.ShapeDtypeStruct((B, DIM), jnp.float32),
        scratch_types=[
            pltpu.VMEM((N_CHUNKS, CHUNK), jnp.int32),
            pltpu.VMEM((CHUNK, DIM), jnp.float32),
            pltpu.SemaphoreType.DMA,
        ],
        compiler_params=pltpu.CompilerParams(use_tc_tiling_on_sc=False),
    )
    def gather_kernel(table_hbm, idx_hbm, out_hbm, idx_v, rows_v, sem):
        wid = lax.axis_index("s") * 2 + lax.axis_index("c")
        chunk0 = wid * N_CHUNKS
        pltpu.sync_copy(idx_hbm.at[pl.ds(chunk0, N_CHUNKS)], idx_v)

        def body(j, _):
            pltpu.async_copy(table_hbm.at[idx_v.at[j]], rows_v, sem).wait()
            pltpu.sync_copy(
                rows_v, out_hbm.at[pl.ds((chunk0 + j) * CHUNK, CHUNK)])
            return 0

        lax.fori_loop(0, N_CHUNKS, body, 0)

    return gather_kernel


_gather = _make_gather()


def kernel(data, ivectors, ovectors):
    idx = data.reshape(B // CHUNK, CHUNK).astype(jnp.int32)
    out = _gather(ivectors, idx)
    return out.reshape(BATCH, HIST, DIM)


# speedup vs baseline: 1.8745x; 1.1136x over previous
import functools

import jax
import jax.numpy as jnp
from jax import lax
from jax.experimental import pallas as pl
from jax.experimental.pallas import tpu as pltpu
from jax.experimental.pallas import tpu_sc as plsc

N_NODES_P1 = 1000001
DIM = 64
BATCH = 16384
HIST = 50

B = BATCH * HIST
NW = 32
B_PER_W = B // NW
CHUNK = 128
N_CHUNKS = B_PER_W // CHUNK
K = 5
N_GROUPS = N_CHUNKS // K
N_PAIRS = N_GROUPS // 2


def _make_gather():
    mesh = plsc.VectorSubcoreMesh(core_axis_name="c", subcore_axis_name="s")

    @functools.partial(
        pl.kernel,
        mesh=mesh,
        out_type=jax.ShapeDtypeStruct((B, DIM), jnp.float32),
        scratch_types=[
            pltpu.VMEM((N_CHUNKS, CHUNK), jnp.int32),
            [[pltpu.VMEM((CHUNK, DIM), jnp.float32) for _ in range(K)]
             for _ in range(2)],
            [[pltpu.SemaphoreType.DMA for _ in range(K)] for _ in range(2)],
            [[pltpu.SemaphoreType.DMA for _ in range(K)] for _ in range(2)],
        ],
        compiler_params=pltpu.CompilerParams(use_tc_tiling_on_sc=False),
    )
    def gather_kernel(table_hbm, idx_hbm, out_hbm, idx_v, rows, gsem, wsem):
        wid = lax.axis_index("s") * 2 + lax.axis_index("c")
        chunk0 = wid * N_CHUNKS
        pltpu.sync_copy(idx_hbm.at[pl.ds(chunk0, N_CHUNKS)], idx_v)

        def fire(group, s, k):
            c = group * K + k
            pltpu.async_copy(table_hbm.at[idx_v.at[c]], rows[s][k],
                             gsem[s][k])

        def drain_gather(group, s, k):
            c = group * K + k
            pltpu.make_async_copy(table_hbm.at[idx_v.at[c]], rows[s][k],
                                  gsem[s][k]).wait()

        for s in range(2):
            for k in range(K):
                fire(s, s, k)

        def pair_body(g2, _):
            for s in range(2):
                g = 2 * g2 + s
                for k in range(K):
                    drain_gather(g, s, k)
                    c = g * K + k
                    pltpu.async_copy(
                        rows[s][k],
                        out_hbm.at[pl.ds((chunk0 + c) * CHUNK, CHUNK)],
                        wsem[s][k])
                for k in range(K):
                    c = g * K + k
                    pltpu.make_async_copy(
                        rows[s][k],
                        out_hbm.at[pl.ds((chunk0 + c) * CHUNK, CHUNK)],
                        wsem[s][k]).wait()

                    @pl.when(g2 < N_PAIRS - 1)
                    def _():
                        fire(g + 2, s, k)
            return 0

        lax.fori_loop(0, N_PAIRS, pair_body, 0)

    return gather_kernel


_gather = _make_gather()


def kernel(data, ivectors, ovectors):
    idx = data.reshape(B // CHUNK, CHUNK).astype(jnp.int32)
    out = _gather(ivectors, idx)
    return out.reshape(BATCH, HIST, DIM)
